# baseline (speedup 1.0000x reference)
import jax
import jax.numpy as jnp
from jax.experimental import pallas as pl
from jax.experimental.pallas import tpu as pltpu

B, S, D = 2, 2048, 1024
HR, HE, HF = 256, 4096, 1024

BS = 512
HEB = 512
NK = HE // HEB
BF = jnp.bfloat16


def _router_kernel(x_ref, wr1_ref, br1_ref, wdom_ref, wgate_ref,
                   dlog_ref, glog_ref):
    pooled = jnp.mean(x_ref[...], axis=1)
    h = jnp.tanh(
        jnp.dot(pooled, wr1_ref[...], preferred_element_type=jnp.float32)
        + br1_ref[...])
    dlog_ref[...] = jnp.dot(h, wdom_ref[...],
                            preferred_element_type=jnp.float32)
    glog_ref[...] = jnp.dot(h, wgate_ref[...],
                            preferred_element_type=jnp.float32)


def _ffn_kernel(conf_ref, x_ref, w1_ref, b1_ref, w2_ref, b2_ref,
                wf1_ref, bf1_ref, wf2_ref, bf2_ref, o_ref, xbf_ref):
    k = pl.program_id(0)
    w1 = w1_ref[...].astype(BF)
    w2 = w2_ref[...].astype(BF)
    for si in range(S // BS):
        sl = pl.ds(si * BS, BS)

        @pl.when(k == 0)
        def _():
            xbf_ref[sl, :] = x_ref[sl, :].astype(BF)

        xs = xbf_ref[sl, :]
        h = jax.nn.gelu(
            jnp.dot(xs, w1, preferred_element_type=jnp.float32)
            + b1_ref[...])
        part = jnp.dot(h.astype(BF), w2,
                       preferred_element_type=jnp.float32)

        @pl.when(k == 0)
        def _():
            o_ref[sl, :] = part

        @pl.when(k > 0)
        def _():
            o_ref[sl, :] += part

    @pl.when(k == NK - 1)
    def _():
        wf1 = wf1_ref[...].astype(BF)
        wf2 = wf2_ref[...].astype(BF)
        c = conf_ref[0, 0]
        for si in range(S // BS):
            sl = pl.ds(si * BS, BS)
            e = o_ref[sl, :] + b2_ref[...]
            t = jnp.tanh(
                jnp.dot(e.astype(BF), wf1,
                        preferred_element_type=jnp.float32)
                + bf1_ref[...])
            f = jnp.dot(t.astype(BF), wf2,
                        preferred_element_type=jnp.float32) + bf2_ref[...]
            o_ref[sl, :] = c * f + x_ref[sl, :]


def _run_ffn(xb, conf_b, W1, b1, W2, b2, Wf1, bf1, Wf2, bf2):
    return pl.pallas_call(
        _ffn_kernel,
        grid=(NK,),
        in_specs=[
            pl.BlockSpec(memory_space=pltpu.SMEM),
            pl.BlockSpec((S, D), lambda k: (0, 0)),
            pl.BlockSpec((D, HEB), lambda k: (0, k)),
            pl.BlockSpec((1, HEB), lambda k: (0, k)),
            pl.BlockSpec((HEB, D), lambda k: (k, 0)),
            pl.BlockSpec((1, D), lambda k: (0, 0)),
            pl.BlockSpec((D, HF), lambda k: (0, 0)),
            pl.BlockSpec((1, HF), lambda k: (0, 0)),
            pl.BlockSpec((HF, D), lambda k: (0, 0)),
            pl.BlockSpec((1, D), lambda k: (0, 0)),
        ],
        out_specs=pl.BlockSpec((S, D), lambda k: (0, 0)),
        out_shape=jax.ShapeDtypeStruct((S, D), jnp.float32),
        scratch_shapes=[pltpu.VMEM((S, D), BF)],
    )(conf_b, xb, W1, b1, W2, b2, Wf1, bf1, Wf2, bf2)


@jax.jit
def kernel(x, Wr1, br1, Wdom, Wop, Wtask, Wgate, Wm1, bm1, Wm2, bm2,
           Wl1, bl1, Wl2, bl2, Wf1, bf1, Wf2, bf2):
    del Wop, Wtask

    dlog, glog = pl.pallas_call(
        _router_kernel,
        out_shape=(
            jax.ShapeDtypeStruct((B, 2), jnp.float32),
            jax.ShapeDtypeStruct((B, 2), jnp.float32),
        ),
    )(x, Wr1, br1.reshape(1, HR), Wdom, Wgate)

    dom = dlog[:, 1] > dlog[:, 0]
    gmax = jnp.max(glog, axis=1, keepdims=True)
    eg = jnp.exp(glog - gmax)
    conf = jnp.where(dom, eg[:, 1], eg[:, 0]) / jnp.sum(eg, axis=1)

    bm1r, bl1r = bm1.reshape(1, HE), bl1.reshape(1, HE)
    bm2r, bl2r = bm2.reshape(1, D), bl2.reshape(1, D)
    bf1r, bf2r = bf1.reshape(1, HF), bf2.reshape(1, D)

    outs = []
    for b in range(B):
        xb = x[b]
        cb = conf[b].reshape(1, 1)
        out_b = jax.lax.cond(
            dom[b],
            lambda xb=xb, cb=cb: _run_ffn(xb, cb, Wl1, bl1r, Wl2, bl2r,
                                          Wf1, bf1r, Wf2, bf2r),
            lambda xb=xb, cb=cb: _run_ffn(xb, cb, Wm1, bm1r, Wm2, bm2r,
                                          Wf1, bf1r, Wf2, bf2r),
        )
        outs.append(out_b)
    return jnp.stack(outs)


# speedup vs baseline: 1.3614x; 1.3614x over previous
import jax
import jax.numpy as jnp
from jax.experimental import pallas as pl
from jax.experimental.pallas import tpu as pltpu

B, S, D = 2, 2048, 1024
HR, HE, HF = 256, 4096, 1024

BS = 512
HEB = 1024
NK = HE // HEB


def _router_kernel(x_ref, wr1_ref, br1_ref, wdom_ref, wgate_ref,
                   dlog_ref, glog_ref):
    pooled = jnp.mean(x_ref[...], axis=1)
    h = jnp.tanh(
        jnp.dot(pooled, wr1_ref[...], preferred_element_type=jnp.float32)
        + br1_ref[...])
    dlog_ref[...] = jnp.dot(h, wdom_ref[...],
                            preferred_element_type=jnp.float32)
    glog_ref[...] = jnp.dot(h, wgate_ref[...],
                            preferred_element_type=jnp.float32)


def _expert_kernel(dom_ref, conf_ref, x_ref, w1m_ref, w1l_ref, b1_ref,
                   w2m_ref, w2l_ref, b2_ref, wf1_ref, bf1_ref, wf2_ref,
                   bf2_ref, o_ref, acc_ref):
    b = pl.program_id(0)
    k = pl.program_id(2)
    d = dom_ref[b]
    xb = x_ref[0]
    w1 = jnp.where(d == 0, w1m_ref[...], w1l_ref[...])
    w2 = jnp.where(d == 0, w2m_ref[...], w2l_ref[...])
    h = jax.nn.gelu(
        jnp.dot(xb, w1, preferred_element_type=jnp.float32)
        + b1_ref[0])
    part = jnp.dot(h, w2, preferred_element_type=jnp.float32)

    @pl.when(k == 0)
    def _():
        acc_ref[...] = part

    @pl.when(k > 0)
    def _():
        acc_ref[...] += part

    @pl.when(k == NK - 1)
    def _():
        e = acc_ref[...] + b2_ref[0]
        t = jnp.tanh(
            jnp.dot(e, wf1_ref[...], preferred_element_type=jnp.float32)
            + bf1_ref[...])
        f = jnp.dot(t, wf2_ref[...],
                    preferred_element_type=jnp.float32) + bf2_ref[...]
        o_ref[0] = conf_ref[b] * f + xb


@jax.jit
def kernel(x, Wr1, br1, Wdom, Wop, Wtask, Wgate, Wm1, bm1, Wm2, bm2,
           Wl1, bl1, Wl2, bl2, Wf1, bf1, Wf2, bf2):
    del Wop, Wtask

    dlog, glog = pl.pallas_call(
        _router_kernel,
        out_shape=(
            jax.ShapeDtypeStruct((B, 2), jnp.float32),
            jax.ShapeDtypeStruct((B, 2), jnp.float32),
        ),
    )(x, Wr1, br1.reshape(1, HR), Wdom, Wgate)

    dom = (dlog[:, 1] > dlog[:, 0]).astype(jnp.int32)
    gmax = jnp.max(glog, axis=1, keepdims=True)
    eg = jnp.exp(glog - gmax)
    conf = jnp.take_along_axis(eg, dom[:, None], axis=1)[:, 0] / jnp.sum(eg, axis=1)

    b1s = jnp.stack([bm1, bl1]).reshape(2, 1, HE)
    b2s = jnp.stack([bm2, bl2]).reshape(2, 1, D)

    def w1_idx(sel):
        def idx(b, s, k, dom, conf):
            return (0, jnp.where(dom[b] == sel, k, 0))
        return idx

    def w2_idx(sel):
        def idx(b, s, k, dom, conf):
            return (jnp.where(dom[b] == sel, k, 0), 0)
        return idx

    grid = (B, S // BS, NK)
    out = pl.pallas_call(
        _expert_kernel,
        grid_spec=pltpu.PrefetchScalarGridSpec(
            num_scalar_prefetch=2,
            grid=grid,
            in_specs=[
                pl.BlockSpec((1, BS, D), lambda b, s, k, dom, conf: (b, s, 0)),
                pl.BlockSpec((D, HEB), w1_idx(0)),
                pl.BlockSpec((D, HEB), w1_idx(1)),
                pl.BlockSpec((1, 1, HEB),
                             lambda b, s, k, dom, conf: (dom[b], 0, k)),
                pl.BlockSpec((HEB, D), w2_idx(0)),
                pl.BlockSpec((HEB, D), w2_idx(1)),
                pl.BlockSpec((1, 1, D),
                             lambda b, s, k, dom, conf: (dom[b], 0, 0)),
                pl.BlockSpec((D, HF), lambda b, s, k, dom, conf: (0, 0)),
                pl.BlockSpec((1, HF), lambda b, s, k, dom, conf: (0, 0)),
                pl.BlockSpec((HF, D), lambda b, s, k, dom, conf: (0, 0)),
                pl.BlockSpec((1, D), lambda b, s, k, dom, conf: (0, 0)),
            ],
            out_specs=pl.BlockSpec((1, BS, D),
                                   lambda b, s, k, dom, conf: (b, s, 0)),
            scratch_shapes=[pltpu.VMEM((BS, D), jnp.float32)],
        ),
        out_shape=jax.ShapeDtypeStruct((B, S, D), jnp.float32),
    )(dom, conf, x, Wm1, Wl1, b1s, Wm2, Wl2, b2s, Wf1, bf1.reshape(1, HF),
      Wf2, bf2.reshape(1, D))
    return out
